# baseline (speedup 1.0000x reference)
import functools

import jax
import jax.numpy as jnp
from jax.experimental import pallas as pl
from jax.experimental.pallas import tpu as pltpu

_SMOOTHING = 0.2
_CONFIDENCE = 1.0 - _SMOOTHING
_ROW_CHUNK = 1024
_NEG = -1e30


def _loss_block_kernel(predt_ref, tgt_ref, out_ref, m8, s8, a8, w8, nc, c):
    i = pl.program_id(1)
    rk, nb = predt_ref.shape
    nt = rk // 8

    @pl.when(i == 0)
    def _init():
        m8[...] = jnp.full((8, nb), _NEG, jnp.float32)
        s8[...] = jnp.zeros((8, nb), jnp.float32)
        a8[...] = jnp.zeros((8, nb), jnp.float32)
        w8[...] = jnp.zeros((8, nb), jnp.float32)

    t = tgt_ref[0]

    def _step(mask_tail):
        x3 = predt_ref[...].reshape(nt, 8, nb)
        rows = (
            jax.lax.broadcasted_iota(jnp.int32, (nt, 8, nb), 0) * 8
            + jax.lax.broadcasted_iota(jnp.int32, (nt, 8, nb), 1)
            + i * rk
        )
        if mask_tail:
            xm = jnp.where(rows < c, x3, _NEG)
        else:
            xm = x3
        m_new = jnp.maximum(m8[...], jnp.max(xm, axis=0))
        s8[...] = s8[...] * jnp.exp(m8[...] - m_new) + jnp.sum(
            jnp.exp(xm - m_new[None]), axis=0
        )
        m8[...] = m_new
        d = rows - t[None] + 1
        center = d == 1
        win = d.astype(jnp.uint32) < 3
        if mask_tail:
            win = jnp.logical_and(win, rows < c)
        a8[...] = a8[...] + jnp.sum(jnp.where(center, x3, 0.0), axis=0)
        w8[...] = w8[...] + jnp.sum(jnp.where(win, x3, 0.0), axis=0)

    @pl.when(i < nc - 1)
    def _fast():
        _step(False)

    @pl.when(i == nc - 1)
    def _tail():
        _step(True)

        m_f = jnp.max(m8[...], axis=0, keepdims=True)
        s_f = jnp.sum(s8[...] * jnp.exp(m8[...] - m_f), axis=0, keepdims=True)
        a = jnp.sum(a8[...], axis=0, keepdims=True)
        w = jnp.sum(w8[...], axis=0, keepdims=True)
        edge = jnp.logical_or(t == 0, t == c - 1)
        s = jnp.where(edge, _SMOOTHING, 0.5 * _SMOOTHING)
        g = (_CONFIDENCE - s) * a + s * w
        out_ref[0] = m_f + jnp.log(s_f) - g


def kernel(pred, target):
    b, c = pred.shape
    predt = pred.T
    rk = _ROW_CHUNK
    nc = pl.cdiv(c, rk)
    nbh = b // 2
    tgt = target.astype(jnp.int32).reshape(2, 1, nbh)

    body = functools.partial(_loss_block_kernel, nc=nc, c=c)
    losses = pl.pallas_call(
        body,
        grid=(2, nc),
        in_specs=[
            pl.BlockSpec((rk, nbh), lambda j, i: (i, j)),
            pl.BlockSpec((1, 1, nbh), lambda j, i: (j, 0, 0)),
        ],
        out_specs=pl.BlockSpec((1, 1, nbh), lambda j, i: (j, 0, 0)),
        out_shape=jax.ShapeDtypeStruct((2, 1, nbh), jnp.float32),
        scratch_shapes=[
            pltpu.VMEM((8, nbh), jnp.float32),
            pltpu.VMEM((8, nbh), jnp.float32),
            pltpu.VMEM((8, nbh), jnp.float32),
            pltpu.VMEM((8, nbh), jnp.float32),
        ],
        compiler_params=pltpu.CompilerParams(
            dimension_semantics=("parallel", "arbitrary"),
            vmem_limit_bytes=64 * 1024 * 1024,
        ),
    )(predt, tgt)
    return jnp.mean(losses.reshape(b))
